# baseline (speedup 1.0000x reference)
import functools

import jax
import jax.numpy as jnp
from jax import lax
from jax.experimental import pallas as pl
from jax.experimental.pallas import tpu as pltpu
from jax.experimental.pallas import tpu_sc as plsc

_V = 100000
_D = 128
_B = 1024
_W2 = 10

_NC, _NS = 2, 16
_NW = _NC * _NS
_BPW = _B // _NW
_IPW = _BPW * _W2
_GCH = 4
_IPC = _IPW // _GCH

_LANES = 16

@functools.cache
def _build_gather_mean():
    mesh = plsc.VectorSubcoreMesh(core_axis_name="c", subcore_axis_name="s")

    @functools.partial(
        pl.kernel,
        mesh=mesh,
        out_type=jax.ShapeDtypeStruct((_B, _D), jnp.float32),
        scratch_types=[
            pltpu.VMEM((_GCH, _IPC), jnp.int32),
            pltpu.VMEM((_IPW, _D), jnp.float32),
            pltpu.VMEM((_BPW, _D), jnp.float32),
            pltpu.SemaphoreType.DMA,
        ],
    )
    def _gather_mean(table_hbm, idx_hbm, h_hbm, idx_v, rows_v, out_v, sem):
        wid = lax.axis_index("s") * _NC + lax.axis_index("c")
        pltpu.sync_copy(idx_hbm.at[wid], idx_v)
        copies = [
            pltpu.async_copy(
                table_hbm.at[idx_v.at[j]], rows_v.at[pl.ds(j * _IPC, _IPC)], sem
            )
            for j in range(_GCH)
        ]
        for c in copies:
            c.wait()

        def body(b, carry):
            base = b * _W2
            for c in range(_D // _LANES):
                sl = pl.ds(c * _LANES, _LANES)
                acc = rows_v[base, sl]
                for w in range(1, _W2):
                    acc = acc + rows_v[base + w, sl]
                out_v[b, sl] = acc * (1.0 / _W2)
            return carry

        lax.fori_loop(0, _BPW, body, 0)
        pltpu.sync_copy(out_v, h_hbm.at[pl.ds(wid * _BPW, _BPW)])

    return _gather_mean


_VB = 1024


def _proj_body(h_ref, w_ref, out_ref):
    out_ref[...] = lax.dot_general(
        h_ref[...].astype(jnp.bfloat16),
        w_ref[...].astype(jnp.bfloat16),
        dimension_numbers=(((1,), (1,)), ((), ())),
        preferred_element_type=jnp.float32,
    )


def kernel(context, embed_in, embed_out):
    idx = context.reshape(_NW, _GCH, _IPC).astype(jnp.int32)
    h = _build_gather_mean()(embed_in, idx)
    grid = (_V + _VB - 1) // _VB
    logits = pl.pallas_call(
        _proj_body,
        grid=(grid,),
        in_specs=[
            pl.BlockSpec((_B, _D), lambda i: (0, 0)),
            pl.BlockSpec((_VB, _D), lambda i: (i, 0)),
        ],
        out_specs=pl.BlockSpec((_B, _VB), lambda i: (0, i)),
        out_shape=jax.ShapeDtypeStruct((_B, _V), jnp.float32),
    )(h, embed_out)
    return logits


# speedup vs baseline: 1.0332x; 1.0332x over previous
import functools

import jax
import jax.numpy as jnp
from jax import lax
from jax.experimental import pallas as pl
from jax.experimental.pallas import tpu as pltpu
from jax.experimental.pallas import tpu_sc as plsc

_V = 100000
_D = 128
_B = 1024
_W2 = 10

_NC, _NS = 2, 16
_NW = _NC * _NS
_BPW = _B // _NW
_IPW = _BPW * _W2
_GCH = 4
_IPC = _IPW // _GCH

_LANES = 16

@functools.cache
def _build_gather_mean():
    mesh = plsc.VectorSubcoreMesh(core_axis_name="c", subcore_axis_name="s")

    @functools.partial(
        pl.kernel,
        mesh=mesh,
        out_type=jax.ShapeDtypeStruct((_B, _D), jnp.float32),
        scratch_types=[
            pltpu.VMEM((_GCH, _IPC), jnp.int32),
            pltpu.VMEM((_IPW, _D), jnp.float32),
            pltpu.VMEM((_BPW, _D), jnp.float32),
            pltpu.SemaphoreType.DMA,
        ],
    )
    def _gather_mean(table_hbm, idx_hbm, h_hbm, idx_v, rows_v, out_v, sem):
        wid = lax.axis_index("s") * _NC + lax.axis_index("c")
        pltpu.sync_copy(idx_hbm.at[wid], idx_v)
        copies = [
            pltpu.async_copy(
                table_hbm.at[idx_v.at[j]], rows_v.at[pl.ds(j * _IPC, _IPC)], sem
            )
            for j in range(_GCH)
        ]
        for c in copies:
            c.wait()

        def body(b, carry):
            base = b * _W2
            for c in range(_D // _LANES):
                sl = pl.ds(c * _LANES, _LANES)
                acc = rows_v[base, sl]
                for w in range(1, _W2):
                    acc = acc + rows_v[base + w, sl]
                out_v[b, sl] = acc * (1.0 / _W2)
            return carry

        lax.fori_loop(0, _BPW, body, 0)
        pltpu.sync_copy(out_v, h_hbm.at[pl.ds(wid * _BPW, _BPW)])

    return _gather_mean


_VB = 2048
_NFULL = _V // _VB
_VTAIL = _V - _NFULL * _VB
_NSTEPS = _NFULL + 1
_NBUF = 4


def _proj_body(h_ref, w_ref, out_hbm, bufs, tail_buf, sems, tail_sem):
    i = pl.program_id(0)
    slot = lax.rem(i, _NBUF)

    @pl.when(i >= _NBUF)
    def _():
        pltpu.make_async_copy(
            bufs.at[slot],
            out_hbm.at[:, pl.ds((i - _NBUF) * _VB, _VB)],
            sems.at[slot],
        ).wait()

    acc = lax.dot_general(
        h_ref[...],
        w_ref[...],
        dimension_numbers=(((1,), (1,)), ((), ())),
        preferred_element_type=jnp.float32,
    )

    @pl.when(i < _NFULL)
    def _():
        bufs[slot] = acc
        pltpu.make_async_copy(
            bufs.at[slot],
            out_hbm.at[:, pl.ds(i * _VB, _VB)],
            sems.at[slot],
        ).start()

    @pl.when(i == _NFULL)
    def _():
        tail_buf[...] = acc[:, :_VTAIL]
        pltpu.make_async_copy(
            tail_buf,
            out_hbm.at[:, pl.ds(_NFULL * _VB, _VTAIL)],
            tail_sem,
        ).start()
        for step in range(_NSTEPS - _NBUF, _NFULL):
            pltpu.make_async_copy(
                bufs.at[step % _NBUF],
                out_hbm.at[:, pl.ds(step * _VB, _VB)],
                sems.at[step % _NBUF],
            ).wait()
        pltpu.make_async_copy(
            tail_buf,
            out_hbm.at[:, pl.ds(_NFULL * _VB, _VTAIL)],
            tail_sem,
        ).wait()


def kernel(context, embed_in, embed_out):
    idx = context.reshape(_NW, _GCH, _IPC).astype(jnp.int32)
    h = _build_gather_mean()(embed_in, idx)
    logits = pl.pallas_call(
        _proj_body,
        grid=(_NSTEPS,),
        in_specs=[
            pl.BlockSpec((_B, _D), lambda i: (0, 0)),
            pl.BlockSpec((_VB, _D), lambda i: (i, 0)),
        ],
        out_specs=pl.BlockSpec(memory_space=pl.ANY),
        out_shape=jax.ShapeDtypeStruct((_B, _V), jnp.float32),
        scratch_shapes=[
            pltpu.VMEM((_NBUF, _B, _VB), jnp.float32),
            pltpu.VMEM((_B, _VTAIL), jnp.float32),
            pltpu.SemaphoreType.DMA((_NBUF,)),
            pltpu.SemaphoreType.DMA,
        ],
    )(h, embed_out)
    return logits


# speedup vs baseline: 3.1689x; 3.0671x over previous
import functools

import jax
import jax.numpy as jnp
from jax import lax
from jax.experimental import pallas as pl
from jax.experimental.pallas import tpu as pltpu
from jax.experimental.pallas import tpu_sc as plsc

_V = 100000
_D = 128
_B = 1024
_W2 = 10

_NC, _NS = 2, 16
_NW = _NC * _NS
_BPW = _B // _NW
_IPW = _BPW * _W2
_GCH = 4
_IPC = _IPW // _GCH

_LANES = 16

@functools.cache
def _build_gather_mean():
    mesh = plsc.VectorSubcoreMesh(core_axis_name="c", subcore_axis_name="s")

    @functools.partial(
        pl.kernel,
        mesh=mesh,
        out_type=jax.ShapeDtypeStruct((_B, _D), jnp.float32),
        scratch_types=[
            pltpu.VMEM((_GCH, _IPC), jnp.int32),
            pltpu.VMEM((_IPW, _D), jnp.float32),
            pltpu.VMEM((_BPW, _D), jnp.float32),
            pltpu.SemaphoreType.DMA,
        ],
    )
    def _gather_mean(table_hbm, idx_hbm, h_hbm, idx_v, rows_v, out_v, sem):
        wid = lax.axis_index("s") * _NC + lax.axis_index("c")
        pltpu.sync_copy(idx_hbm.at[wid], idx_v)
        copies = [
            pltpu.async_copy(
                table_hbm.at[idx_v.at[j]], rows_v.at[pl.ds(j * _IPC, _IPC)], sem
            )
            for j in range(_GCH)
        ]
        for c in copies:
            c.wait()

        def body(b, carry):
            base = b * _W2
            for c in range(_D // _LANES):
                sl = pl.ds(c * _LANES, _LANES)
                acc = rows_v[base, sl]
                for w in range(1, _W2):
                    acc = acc + rows_v[base + w, sl]
                out_v[b, sl] = acc * (1.0 / _W2)
            return carry

        lax.fori_loop(0, _BPW, body, 0)
        pltpu.sync_copy(out_v, h_hbm.at[pl.ds(wid * _BPW, _BPW)])

    return _gather_mean


_VB = 2048
_NFULL = _V // _VB
_VTAIL = _V - _NFULL * _VB
_NSTEPS = _NFULL + 1
_NBUF = 4


def _proj_body(w_ref, h_ref, out_ref):
    out_ref[...] = lax.dot_general(
        w_ref[...],
        h_ref[...],
        dimension_numbers=(((1,), (1,)), ((), ())),
        preferred_element_type=jnp.float32,
    )


def kernel(context, embed_in, embed_out):
    idx = context.reshape(_NW, _GCH, _IPC).astype(jnp.int32)
    h = _build_gather_mean()(embed_in, idx)
    logits_t = pl.pallas_call(
        _proj_body,
        grid=(_NSTEPS,),
        in_specs=[
            pl.BlockSpec((_VB, _D), lambda i: (i, 0)),
            pl.BlockSpec((_B, _D), lambda i: (0, 0)),
        ],
        out_specs=pl.BlockSpec((_VB, _B), lambda i: (i, 0)),
        out_shape=jax.ShapeDtypeStruct((_V, _B), jnp.float32),
    )(embed_out, h)
    return logits_t.T


# speedup vs baseline: 3.2360x; 1.0212x over previous
import functools

import jax
import jax.numpy as jnp
from jax import lax
from jax.experimental import pallas as pl
from jax.experimental.pallas import tpu as pltpu
from jax.experimental.pallas import tpu_sc as plsc

_V = 100000
_D = 128
_B = 1024
_W2 = 10

_NC, _NS = 2, 16
_NW = _NC * _NS
_BPW = _B // _NW
_IPW = _BPW * _W2
_GCH = 4
_IPC = _IPW // _GCH

_LANES = 16

@functools.cache
def _build_gather_mean():
    mesh = plsc.VectorSubcoreMesh(core_axis_name="c", subcore_axis_name="s")

    @functools.partial(
        pl.kernel,
        mesh=mesh,
        out_type=jax.ShapeDtypeStruct((_B, _D), jnp.float32),
        scratch_types=[
            pltpu.VMEM((_GCH, _IPC), jnp.int32),
            pltpu.VMEM((_IPW, _D), jnp.float32),
            pltpu.VMEM((_BPW, _D), jnp.float32),
            pltpu.SemaphoreType.DMA,
        ],
    )
    def _gather_mean(table_hbm, idx_hbm, h_hbm, idx_v, rows_v, out_v, sem):
        wid = lax.axis_index("s") * _NC + lax.axis_index("c")
        pltpu.sync_copy(idx_hbm.at[wid], idx_v)
        copies = [
            pltpu.async_copy(
                table_hbm.at[idx_v.at[j]], rows_v.at[pl.ds(j * _IPC, _IPC)], sem
            )
            for j in range(_GCH)
        ]
        for c in copies:
            c.wait()

        def body(b, carry):
            base = b * _W2
            for c in range(_D // _LANES):
                sl = pl.ds(c * _LANES, _LANES)
                acc = rows_v[base, sl]
                for w in range(1, _W2):
                    acc = acc + rows_v[base + w, sl]
                out_v[b, sl] = acc * (1.0 / _W2)
            return carry

        lax.fori_loop(0, _BPW, body, 0)
        pltpu.sync_copy(out_v, h_hbm.at[pl.ds(wid * _BPW, _BPW)])

    return _gather_mean


_VB = 4096
_NFULL = _V // _VB
_VTAIL = _V - _NFULL * _VB
_NSTEPS = _NFULL + 1
_NBUF = 4


def _proj_body(w_ref, h_ref, out_ref):
    out_ref[...] = lax.dot_general(
        w_ref[...],
        h_ref[...],
        dimension_numbers=(((1,), (1,)), ((), ())),
        preferred_element_type=jnp.float32,
    )


def kernel(context, embed_in, embed_out):
    idx = context.reshape(_NW, _GCH, _IPC).astype(jnp.int32)
    h = _build_gather_mean()(embed_in, idx)
    logits_t = pl.pallas_call(
        _proj_body,
        grid=(_NSTEPS,),
        in_specs=[
            pl.BlockSpec((_VB, _D), lambda i: (i, 0)),
            pl.BlockSpec((_B, _D), lambda i: (0, 0)),
        ],
        out_specs=pl.BlockSpec((_VB, _B), lambda i: (i, 0)),
        out_shape=jax.ShapeDtypeStruct((_V, _B), jnp.float32),
    )(embed_out, h)
    return logits_t.T
